# Initial kernel scaffold; baseline (speedup 1.0000x reference)
import jax
import jax.numpy as jnp
from jax.experimental import pallas as pl


def kernel(actions):
    raise NotImplementedError("write your pallas kernel here")


# speedup vs baseline: 20992.3255x; 20992.3255x over previous
import functools

import jax
import jax.numpy as jnp
from jax import lax
from jax.experimental import pallas as pl
from jax.experimental.pallas import tpu as pltpu
from jax.experimental.pallas import tpu_sc as plsc

_N = 33554432
_NUM_WORKERS = 32
_PER_W = _N // _NUM_WORKERS
_CH = 16384
_NB = 2
_NOUTER = _PER_W // (_CH * _NB)

_mesh = plsc.VectorSubcoreMesh(core_axis_name="c", subcore_axis_name="s")


@functools.partial(
    pl.kernel,
    mesh=_mesh,
    out_type=jax.ShapeDtypeStruct((_N,), jnp.int32),
    scratch_types=[
        pltpu.VMEM((_NB, _CH), jnp.float32),
        pltpu.VMEM((_NB, _CH), jnp.int32),
        pltpu.SemaphoreType.DMA,
        pltpu.SemaphoreType.DMA,
        pltpu.SemaphoreType.DMA,
        pltpu.SemaphoreType.DMA,
    ],
)
def _discretize_sc(x_hbm, o_hbm, in_v, out_v, is0, is1, os0, os1):
    isems = (is0, is1)
    osems = (os0, os1)
    wid = lax.axis_index("s") * 2 + lax.axis_index("c")
    base = wid * _PER_W

    for b in range(_NB):
        pltpu.async_copy(
            x_hbm.at[pl.ds(base + b * _CH, _CH)], in_v.at[b], isems[b]
        )

    def outer(g2, _):
        for b in range(_NB):
            off = base + (g2 * _NB + b) * _CH
            pltpu.make_async_copy(
                x_hbm.at[pl.ds(off, _CH)], in_v.at[b], isems[b]
            ).wait()

            @pl.when(g2 > 0)
            def _wait_out():
                pltpu.make_async_copy(
                    out_v.at[b], o_hbm.at[pl.ds(off, _CH)], osems[b]
                ).wait()

            @plsc.parallel_loop(0, _CH, step=16, unroll=8)
            def _compute(i):
                x = in_v[b, pl.ds(i, 16)]
                y = x * 128.0
                y = jnp.minimum(jnp.maximum(y, -129.0), 256.0)
                t = y.astype(jnp.int32)
                f = t.astype(jnp.float32)
                r = jnp.where(y < f, t - 1, t) + 128
                r = jnp.minimum(jnp.maximum(r, 0), 255)
                out_v[b, pl.ds(i, 16)] = r

            pltpu.async_copy(out_v.at[b], o_hbm.at[pl.ds(off, _CH)], osems[b])

            @pl.when(g2 < _NOUTER - 1)
            def _next_in():
                pltpu.async_copy(
                    x_hbm.at[pl.ds(off + _NB * _CH, _CH)], in_v.at[b], isems[b]
                )

        return _

    lax.fori_loop(0, _NOUTER, outer, None)
    for b in range(_NB):
        pltpu.make_async_copy(
            out_v.at[b], o_hbm.at[pl.ds(base, _CH)], osems[b]
        ).wait()


def kernel(actions):
    return _discretize_sc(actions)


# speedup vs baseline: 24413.9998x; 1.1630x over previous
import functools

import jax
import jax.numpy as jnp
from jax import lax
from jax.experimental import pallas as pl
from jax.experimental.pallas import tpu as pltpu
from jax.experimental.pallas import tpu_sc as plsc

_N = 33554432
_NUM_WORKERS = 32
_PER_W = _N // _NUM_WORKERS
_CH = 16384
_NB = 2
_NOUTER = _PER_W // (_CH * _NB)

_mesh = plsc.VectorSubcoreMesh(core_axis_name="c", subcore_axis_name="s")


@functools.partial(
    pl.kernel,
    mesh=_mesh,
    out_type=jax.ShapeDtypeStruct((_N,), jnp.int32),
    scratch_types=[
        pltpu.VMEM((_NB, _CH), jnp.float32),
        pltpu.VMEM((_NB, _CH), jnp.int32),
        pltpu.SemaphoreType.DMA,
        pltpu.SemaphoreType.DMA,
        pltpu.SemaphoreType.DMA,
        pltpu.SemaphoreType.DMA,
    ],
)
def _discretize_sc(x_hbm, o_hbm, in_v, out_v, is0, is1, os0, os1):
    isems = (is0, is1)
    osems = (os0, os1)
    wid = lax.axis_index("s") * 2 + lax.axis_index("c")
    base = wid * _PER_W

    for b in range(_NB):
        pltpu.async_copy(
            x_hbm.at[pl.ds(base + b * _CH, _CH)], in_v.at[b], isems[b]
        )

    def outer(g2, _):
        for b in range(_NB):
            off = base + (g2 * _NB + b) * _CH
            pltpu.make_async_copy(
                x_hbm.at[pl.ds(off, _CH)], in_v.at[b], isems[b]
            ).wait()

            @pl.when(g2 > 0)
            def _wait_out():
                pltpu.make_async_copy(
                    out_v.at[b], o_hbm.at[pl.ds(off, _CH)], osems[b]
                ).wait()

            @plsc.parallel_loop(0, _CH, step=16, unroll=8)
            def _compute(i):
                x = in_v[b, pl.ds(i, 16)]
                y = x * 128.0
                y = jnp.minimum(jnp.maximum(y, -128.0), 127.0)
                t = y.astype(jnp.int32)
                f = t.astype(jnp.float32)
                out_v[b, pl.ds(i, 16)] = jnp.where(y < f, t - 1, t) + 128

            pltpu.async_copy(out_v.at[b], o_hbm.at[pl.ds(off, _CH)], osems[b])

            @pl.when(g2 < _NOUTER - 1)
            def _next_in():
                pltpu.async_copy(
                    x_hbm.at[pl.ds(off + _NB * _CH, _CH)], in_v.at[b], isems[b]
                )

        return _

    lax.fori_loop(0, _NOUTER, outer, None)
    for b in range(_NB):
        pltpu.make_async_copy(
            out_v.at[b], o_hbm.at[pl.ds(base, _CH)], osems[b]
        ).wait()


def kernel(actions):
    return _discretize_sc(actions)
